# baseline (speedup 1.0000x reference)
import math

import numpy as np
import jax
import jax.numpy as jnp
from jax import lax
from jax.experimental import pallas as pl
from jax.experimental.pallas import tpu as pltpu
from jax.experimental.pallas import tpu_sc as plsc

VOCAB = 1000000
EMB = 64
MAX_LEN = 512
BATCH = 4096
SEQ = 200
SCALE = math.sqrt(EMB)

NC = 2
NS = 16
L = 16
NW = NC * NS
WB = BATCH // NW
JB = 8
NBLK = SEQ // JB
NG = WB // L
NR = 2
NO = 2


def _pos_embedding_np():
    rng = np.exp(-np.arange(0, EMB, 2, dtype=np.float64) * math.log(10000) / EMB)
    pos = np.arange(0, MAX_LEN, dtype=np.float64).reshape(MAX_LEN, 1)
    pe = np.zeros((MAX_LEN, EMB), dtype=np.float32)
    pe[:, 0::2] = np.sin(pos * rng).astype(np.float32)
    pe[:, 1::2] = np.cos(pos * rng).astype(np.float32)
    return pe[:SEQ]


_PE = _pos_embedding_np()


_DNUMS = lax.GatherDimensionNumbers(
    offset_dims=(), collapsed_slice_dims=(0,), start_index_map=(0,))
_PIB = lax.GatherScatterMode.PROMISE_IN_BOUNDS


def _perm(vec, idx):
    return lax.gather(vec, idx.reshape(L, 1), _DNUMS, (1,), mode=_PIB)


def _sc_body(tok_t, pe_hbm, wtab, out_hbm, pe_v, tokv, widx, selv, rowbuf,
             otile, skew, tsem, gsem, ssem):
    wid = lax.axis_index("s") * NC + lax.axis_index("c")
    i0 = pl.multiple_of(wid * WB, WB)

    iota = lax.iota(jnp.int32, L)
    i128 = iota * (2 * EMB)

    pltpu.sync_copy(pe_hbm, pe_v)

    def tok_src(jb):
        return tok_t.at[pl.ds(pl.multiple_of(JB * jb, JB), JB),
                        pl.ds(i0, WB)]

    def build_block(tb):
        for jj in range(JB):
            for g in range(NG):
                t = tokv[tb, jj, pl.ds(16 * g, L)]
                widx[tb, jj, pl.ds(16 * g, L)] = lax.shift_right_logical(t, 1)
                selv[tb, jj, pl.ds(16 * g, L)] = lax.bitwise_and(
                    t, jnp.int32(1))

    def start_gather(tb, jj, rb):
        pltpu.async_copy(wtab.at[widx.at[tb, jj]], rowbuf.at[rb],
                         gsem.at[rb])

    def wait_gather(tb, jj, rb):
        pltpu.make_async_copy(wtab.at[widx.at[tb, jj]], rowbuf.at[rb],
                              gsem.at[rb]).wait()

    def out_dst(j):
        return out_hbm.at[j, :, pl.ds(i0, WB)]

    def start_store(j, ob):
        pltpu.async_copy(otile.at[ob], out_dst(j), ssem.at[ob])

    def wait_store(j, ob):
        pltpu.make_async_copy(otile.at[ob], out_dst(j), ssem.at[ob]).wait()

    pltpu.sync_copy(tok_src(0), tokv.at[0])
    build_block(0)
    for p in range(NR - 1):
        start_gather(0, p, p)
    pltpu.async_copy(tok_src(1), tokv.at[1], tsem)

    def jb_body(jb, carry):
        tb = lax.rem(jb, 2)
        tb1 = lax.rem(jb + 1, 2)

        @pl.when(jb + 1 < NBLK)
        def _():
            pltpu.make_async_copy(tok_src(jb + 1), tokv.at[tb1], tsem).wait()
            build_block(tb1)

            @pl.when(jb + 2 < NBLK)
            def _():
                pltpu.async_copy(tok_src(jb + 2), tokv.at[tb], tsem)

        def jj_body(jj, c2):
            j = JB * jb + jj
            rb = lax.rem(jj, NR)
            ob = lax.rem(jj, NO)

            wait_gather(tb, jj, rb)

            rb3 = lax.rem(jj + NR - 1, NR)

            @pl.when(jj + NR - 1 < JB)
            def _():
                start_gather(tb, jj + NR - 1, rb3)

            @pl.when((jj + NR - 1 >= JB) & (j + NR - 1 < SEQ))
            def _():
                start_gather(tb1, jj + NR - 1 - JB, rb3)

            @pl.when(j >= NO)
            def _():
                wait_store(j - NO, ob)

            def blk_body(blk, c3):
                for k in range(L):
                    base = (16 * blk + k) * (2 * EMB)
                    rk = lax.rem(iota + k, L)
                    for h in range(NG):
                        vreg = rowbuf[rb, 16 * blk + k, pl.ds(16 * h, L)]
                        skew[pl.ds(base + 16 * h, L)] = _perm(vreg, rk)
                return c3

            lax.fori_loop(0, NG, blk_body, 0)

            s64 = [selv[tb, jj, pl.ds(16 * g, L)] * EMB for g in range(NG)]
            pb = [i128 + s64[g] + 16 * (2 * EMB) * g for g in range(NG)]
            pev = [pe_v[j, pl.ds(16 * q, L)] for q in range(EMB // L)]

            for q in range(EMB // L):
                def m_body(m, c4, q=q):
                    d = 16 * q + m
                    qm = lax.rem(m + L - iota, L) + 16 * q
                    pe_b = _perm(pev[q], jnp.full((L,), 0, jnp.int32) + m)
                    for g in range(NG):
                        v = plsc.load_gather(skew, [pb[g] + qm])
                        otile[ob, d, pl.ds(16 * g, L)] = v * SCALE + pe_b
                    return c4

                lax.fori_loop(0, L, m_body, 0, unroll=2)

            start_store(j, ob)
            return c2

        lax.fori_loop(0, JB, jj_body, 0)
        return carry

    lax.fori_loop(0, NBLK, jb_body, 0)

    for k in range(NO):
        wait_store(SEQ - NO + k, (SEQ - NO + k) % NO)


def kernel(tokens, embedding_weight):
    tok_t = tokens.astype(jnp.int32).T
    wtab = embedding_weight.reshape(VOCAB // 2, 2 * EMB)
    pe = jnp.asarray(_PE)
    mesh = plsc.VectorSubcoreMesh(
        core_axis_name="c", subcore_axis_name="s", num_cores=NC,
        num_subcores=NS)
    k = pl.kernel(
        _sc_body,
        out_type=jax.ShapeDtypeStruct((SEQ, EMB, BATCH), jnp.float32),
        mesh=mesh,
        scratch_types=[
            pltpu.VMEM((SEQ, EMB), jnp.float32),
            pltpu.VMEM((2, JB, WB), jnp.int32),
            pltpu.VMEM((2, JB, WB), jnp.int32),
            pltpu.VMEM((2, JB, WB), jnp.int32),
            pltpu.VMEM((NR, WB, 2 * EMB), jnp.float32),
            pltpu.VMEM((NO, EMB, WB), jnp.float32),
            pltpu.VMEM((WB * 2 * EMB,), jnp.float32),
            pltpu.SemaphoreType.DMA,
            pltpu.SemaphoreType.DMA((NR,)),
            pltpu.SemaphoreType.DMA((NO,)),
        ],
        compiler_params=pltpu.CompilerParams(needs_layout_passes=False),
    )
    out_p = k(tok_t, pe, wtab)
    return out_p.transpose(2, 0, 1)


# speedup vs baseline: 1.4729x; 1.4729x over previous
import math

import numpy as np
import jax
import jax.numpy as jnp
from jax import lax
from jax.experimental import pallas as pl
from jax.experimental.pallas import tpu as pltpu
from jax.experimental.pallas import tpu_sc as plsc

VOCAB = 1000000
EMB = 64
MAX_LEN = 512
BATCH = 4096
SEQ = 200
SCALE = math.sqrt(EMB)

NC = 2
NS = 16
L = 16
NW = NC * NS
ROWS = BATCH * SEQ
RPW = ROWS // NW
CG = 256
NCH = RPW // CG
NB = 4


def _pos_embedding_np():
    rng = np.exp(-np.arange(0, EMB, 2, dtype=np.float64) * math.log(10000) / EMB)
    pos = np.arange(0, MAX_LEN, dtype=np.float64).reshape(MAX_LEN, 1)
    pe = np.zeros((MAX_LEN, EMB), dtype=np.float32)
    pe[:, 0::2] = np.sin(pos * rng).astype(np.float32)
    pe[:, 1::2] = np.cos(pos * rng).astype(np.float32)
    return pe[:SEQ]


_PE = _pos_embedding_np()


def _sc_body(tokens_hbm, pe_hbm, table_hbm, out_hbm, pe_v, tok_v, row_v,
             gsem, ssem):
    wid = lax.axis_index("s") * NC + lax.axis_index("c")
    r0 = pl.multiple_of(wid * RPW, 8)

    pltpu.sync_copy(pe_hbm, pe_v)
    pltpu.sync_copy(tokens_hbm.at[pl.ds(r0, RPW)], tok_v)

    def start_gather(c, s):
        pltpu.async_copy(table_hbm.at[tok_v.at[pl.ds(CG * c, CG)]],
                         row_v.at[s], gsem.at[s])

    def wait_gather(c, s):
        pltpu.make_async_copy(table_hbm.at[tok_v.at[pl.ds(CG * c, CG)]],
                              row_v.at[s], gsem.at[s]).wait()

    def out_dst(c):
        return out_hbm.at[pl.ds(r0 + CG * c, CG)]

    def start_store(c, s):
        pltpu.async_copy(row_v.at[s], out_dst(c), ssem.at[s])

    def wait_store(c, s):
        pltpu.make_async_copy(row_v.at[s], out_dst(c), ssem.at[s]).wait()

    start_gather(0, 0)
    start_gather(1, 1)

    def c_body(c, carry):
        s = lax.rem(c, NB)
        s2 = lax.rem(c + 2, NB)

        @pl.when(c + 2 < NCH)
        def _():
            @pl.when(c >= 2)
            def _():
                wait_store(c - 2, s2)

            start_gather(c + 2, s2)

        wait_gather(c, s)

        p0 = lax.rem(r0 + CG * c, SEQ)

        def r_body(r, p):
            for qq in range(EMB // L):
                row_v[s, r, pl.ds(qq * L, L)] = (
                    row_v[s, r, pl.ds(qq * L, L)] * SCALE
                    + pe_v[p, pl.ds(qq * L, L)])
            p = p + 1
            return lax.select(p >= SEQ, p - SEQ, p)

        lax.fori_loop(0, CG, r_body, p0, unroll=8)
        start_store(c, s)
        return carry

    lax.fori_loop(0, NCH, c_body, 0)

    for k in range(NB):
        c = NCH - NB + k
        wait_store(c, lax.rem(jnp.int32(c), NB))


def kernel(tokens, embedding_weight):
    tokens_flat = tokens.astype(jnp.int32).reshape(ROWS)
    pe = jnp.asarray(_PE)
    mesh = plsc.VectorSubcoreMesh(
        core_axis_name="c", subcore_axis_name="s", num_cores=NC,
        num_subcores=NS)
    k = pl.kernel(
        _sc_body,
        out_type=jax.ShapeDtypeStruct((ROWS, EMB), jnp.float32),
        mesh=mesh,
        scratch_types=[
            pltpu.VMEM((SEQ, EMB), jnp.float32),
            pltpu.VMEM((RPW,), jnp.int32),
            pltpu.VMEM((NB, CG, EMB), jnp.float32),
            pltpu.SemaphoreType.DMA((NB,)),
            pltpu.SemaphoreType.DMA((NB,)),
        ],
        compiler_params=pltpu.CompilerParams(use_tc_tiling_on_sc=False),
    )
    out = k(tokens_flat, pe, embedding_weight)
    return out.reshape(BATCH, SEQ, EMB)
